# Initial kernel scaffold; baseline (speedup 1.0000x reference)
import jax
import jax.numpy as jnp
from jax.experimental import pallas as pl


def kernel(x, edge_index, embed_W, embed_b, conv_W1, conv_b1, conv_W2, conv_b2, cls_W, cls_b):
    raise NotImplementedError("write your pallas kernel here")


# speedup vs baseline: 11.8664x; 11.8664x over previous
import functools

import jax
import jax.numpy as jnp
from jax import lax
from jax.experimental import pallas as pl
from jax.experimental.pallas import tpu as pltpu
from jax.experimental.pallas import tpu_sc as plsc

N = 10000
FEAT = 128
OUTD = 64
NC, NS = 2, 16
NW = NC * NS
CHUNK = 128
ACC_ROWS = 10240
ZROWS = ACC_ROWS // NS
DEGW = 16

_MESH = dict(core_axis_name="c", subcore_axis_name="s",
             num_cores=NC, num_subcores=NS)


def _deg_body(dst_hbm, ones_hbm, zeros_hbm, out_hbm, dst_v, ones_v, acc):
    cid = lax.axis_index("c")
    sid = lax.axis_index("s")
    w = cid * NS + sid
    nchunks = dst_hbm.shape[1]
    pltpu.sync_copy(dst_hbm.at[w], dst_v)
    pltpu.sync_copy(ones_hbm, ones_v)
    for z in range(ZROWS // 128):
        pltpu.sync_copy(zeros_hbm, acc.at[pl.ds(sid * ZROWS + z * 128, 128)])
    plsc.subcore_barrier()

    def step(j, carry):
        pltpu.sync_copy(ones_v, acc.at[dst_v.at[j]], add=True)
        return carry

    lax.fori_loop(0, nchunks, step, 0)
    plsc.subcore_barrier()
    pltpu.sync_copy(acc.at[pl.ds(sid * ZROWS, ZROWS)],
                    out_hbm.at[cid, pl.ds(sid * ZROWS, ZROWS)])


def _sc_degree(dst3, ones, zeros):
    kern = functools.partial(
        pl.kernel,
        out_type=jax.ShapeDtypeStruct((NC, ACC_ROWS, FEAT), jnp.float32),
        mesh=plsc.VectorSubcoreMesh(**_MESH),
        scratch_types=[
            pltpu.VMEM(dst3.shape[1:], jnp.int32),
            pltpu.VMEM((CHUNK, FEAT), jnp.float32),
            pltpu.VMEM_SHARED((ACC_ROWS, FEAT), jnp.float32),
        ],
    )(_deg_body)
    return kern(dst3, ones, zeros)


def _agg_body(g_hbm, src_hbm, dst_hbm, zeros_hbm, out_hbm,
              src_v, dst_v, rows_v, acc, sem):
    cid = lax.axis_index("c")
    sid = lax.axis_index("s")
    w = cid * NS + sid
    nchunks = src_hbm.shape[1]
    pltpu.sync_copy(src_hbm.at[w], src_v)
    pltpu.sync_copy(dst_hbm.at[w], dst_v)
    for z in range(ZROWS // 128):
        pltpu.sync_copy(zeros_hbm, acc.at[pl.ds(sid * ZROWS + z * 128, 128)])
    plsc.subcore_barrier()

    def step(j, carry):
        pltpu.async_copy(g_hbm.at[src_v.at[j]], rows_v, sem).wait()
        pltpu.sync_copy(rows_v, acc.at[dst_v.at[j]], add=True)
        return carry

    lax.fori_loop(0, nchunks, step, 0)
    plsc.subcore_barrier()
    pltpu.sync_copy(acc.at[pl.ds(sid * ZROWS, ZROWS)],
                    out_hbm.at[cid, pl.ds(sid * ZROWS, ZROWS)])


def _sc_aggregate(g, src3, dst3, zeros):
    kern = functools.partial(
        pl.kernel,
        out_type=jax.ShapeDtypeStruct((NC, ACC_ROWS, FEAT), jnp.float32),
        mesh=plsc.VectorSubcoreMesh(**_MESH),
        scratch_types=[
            pltpu.VMEM(src3.shape[1:], jnp.int32),
            pltpu.VMEM(dst3.shape[1:], jnp.int32),
            pltpu.VMEM((CHUNK, FEAT), jnp.float32),
            pltpu.VMEM_SHARED((ACC_ROWS, FEAT), jnp.float32),
            pltpu.SemaphoreType.DMA,
        ],
    )(_agg_body)
    return kern(g, src3, dst3, zeros)


ROWB = 2000


def _dinv_from(dp_ref):
    deg = dp_ref[0, :, 0:1] + dp_ref[1, :, 0:1] + 1.0
    return lax.rsqrt(deg)


def _t1_body(x_ref, we_ref, be_ref, w1_ref, dp_ref, h0_ref, g1_ref):
    h0 = jnp.maximum(
        jnp.dot(x_ref[...], we_ref[...], preferred_element_type=jnp.float32)
        + be_ref[...], 0.0)
    h0_ref[...] = h0
    g1_ref[...] = (_dinv_from(dp_ref) *
                   jnp.dot(h0, w1_ref[...], preferred_element_type=jnp.float32))


def _tc_embed(x, embed_W, embed_b, conv_W1, dp):
    grid = (N // ROWB,)
    return pl.pallas_call(
        _t1_body,
        grid=grid,
        in_specs=[
            pl.BlockSpec((ROWB, FEAT), lambda i: (i, 0)),
            pl.BlockSpec((FEAT, FEAT), lambda i: (0, 0)),
            pl.BlockSpec((1, FEAT), lambda i: (0, 0)),
            pl.BlockSpec((FEAT, FEAT), lambda i: (0, 0)),
            pl.BlockSpec((NC, ROWB, FEAT), lambda i: (0, i, 0)),
        ],
        out_specs=[
            pl.BlockSpec((ROWB, FEAT), lambda i: (i, 0)),
            pl.BlockSpec((ROWB, FEAT), lambda i: (i, 0)),
        ],
        out_shape=[
            jax.ShapeDtypeStruct((N, FEAT), jnp.float32),
            jax.ShapeDtypeStruct((N, FEAT), jnp.float32),
        ],
    )(x, embed_W, embed_b.reshape(1, FEAT), conv_W1, dp)


def _t2_body(p_ref, g_ref, dp_ref, b_ref, w_ref, h_ref, gn_ref):
    dinv = _dinv_from(dp_ref)
    agg = p_ref[0] + p_ref[1] + g_ref[...]
    h = jnp.maximum(dinv * agg + b_ref[...], 0.0)
    h_ref[...] = h
    gn_ref[...] = dinv * jnp.dot(h, w_ref[...],
                                 preferred_element_type=jnp.float32)


def _tc_mid(p, g, dp, b, W_next):
    grid = (N // ROWB,)
    return pl.pallas_call(
        _t2_body,
        grid=grid,
        in_specs=[
            pl.BlockSpec((NC, ROWB, FEAT), lambda i: (0, i, 0)),
            pl.BlockSpec((ROWB, FEAT), lambda i: (i, 0)),
            pl.BlockSpec((NC, ROWB, FEAT), lambda i: (0, i, 0)),
            pl.BlockSpec((1, FEAT), lambda i: (0, 0)),
            pl.BlockSpec((FEAT, FEAT), lambda i: (0, 0)),
        ],
        out_specs=[
            pl.BlockSpec((ROWB, FEAT), lambda i: (i, 0)),
            pl.BlockSpec((ROWB, FEAT), lambda i: (i, 0)),
        ],
        out_shape=[
            jax.ShapeDtypeStruct((N, FEAT), jnp.float32),
            jax.ShapeDtypeStruct((N, FEAT), jnp.float32),
        ],
    )(p, g, dp, b.reshape(1, FEAT), W_next)


def _t3_body(p_ref, g_ref, dp_ref, b_ref, h0_ref, h1_ref, cw_ref, cb_ref,
             out_ref):
    dinv = _dinv_from(dp_ref)
    h2 = jnp.maximum(dinv * (p_ref[0] + p_ref[1] + g_ref[...]) + b_ref[...],
                     0.0)
    cw = cw_ref[...]
    out = jnp.dot(h0_ref[...], cw[0:FEAT], preferred_element_type=jnp.float32)
    out += jnp.dot(h1_ref[...], cw[FEAT:2 * FEAT],
                   preferred_element_type=jnp.float32)
    out += jnp.dot(h2, cw[2 * FEAT:3 * FEAT],
                   preferred_element_type=jnp.float32)
    out_ref[...] = out + cb_ref[...]


def _tc_final(p, g, dp, b, h0, h1, cls_W, cls_b):
    grid = (N // ROWB,)
    return pl.pallas_call(
        _t3_body,
        grid=grid,
        in_specs=[
            pl.BlockSpec((NC, ROWB, FEAT), lambda i: (0, i, 0)),
            pl.BlockSpec((ROWB, FEAT), lambda i: (i, 0)),
            pl.BlockSpec((NC, ROWB, FEAT), lambda i: (0, i, 0)),
            pl.BlockSpec((1, FEAT), lambda i: (0, 0)),
            pl.BlockSpec((ROWB, FEAT), lambda i: (i, 0)),
            pl.BlockSpec((ROWB, FEAT), lambda i: (i, 0)),
            pl.BlockSpec((3 * FEAT, OUTD), lambda i: (0, 0)),
            pl.BlockSpec((1, OUTD), lambda i: (0, 0)),
        ],
        out_specs=pl.BlockSpec((ROWB, OUTD), lambda i: (i, 0)),
        out_shape=jax.ShapeDtypeStruct((N, OUTD), jnp.float32),
    )(p, g, dp, b.reshape(1, FEAT), h0, h1, cls_W, cls_b.reshape(1, OUTD))


def kernel(x, edge_index, embed_W, embed_b, conv_W1, conv_b1,
           conv_W2, conv_b2, cls_W, cls_b):
    src = edge_index[0].astype(jnp.int32)
    dst = edge_index[1].astype(jnp.int32)
    e = src.shape[0]
    ept = -(-e // (NW * CHUNK)) * CHUNK
    pad = NW * ept - e
    src_p = jnp.concatenate([src, jnp.zeros((pad,), jnp.int32)])
    dst_p = jnp.concatenate([dst, jnp.full((pad,), N, jnp.int32)])
    src3 = src_p.reshape(NW, ept // CHUNK, CHUNK)
    dst3 = dst_p.reshape(NW, ept // CHUNK, CHUNK)

    ones128 = jnp.ones((CHUNK, FEAT), jnp.float32)
    zeros128 = jnp.zeros((128, FEAT), jnp.float32)

    dp = _sc_degree(dst3, ones128, zeros128)
    h0, g1 = _tc_embed(x, embed_W, embed_b, conv_W1, dp)
    p1 = _sc_aggregate(g1, src3, dst3, zeros128)
    h1, g2 = _tc_mid(p1, g1, dp, conv_b1, conv_W2)
    p2 = _sc_aggregate(g2, src3, dst3, zeros128)
    out = _tc_final(p2, g2, dp, conv_b2, h0, h1, cls_W, cls_b)
    return out
